# Initial kernel scaffold; baseline (speedup 1.0000x reference)
import jax
import jax.numpy as jnp
from jax.experimental import pallas as pl


def kernel(model_predict, target, training_masks):
    raise NotImplementedError("write your pallas kernel here")


# speedup vs baseline: 6.7965x; 6.7965x over previous
import jax
import jax.numpy as jnp
import numpy as np
from jax.experimental import pallas as pl
from jax.experimental.pallas import tpu as pltpu

_EPS = float(np.spacing(1.0))
_INT_MIN = np.int32(-2147483648)
_ROWS = 64


def _float_key(t):
    ti = jax.lax.bitcast_convert_type(t, jnp.int32)
    return jnp.where(ti < 0, _INT_MIN - ti, ti)


def _body(pred_ref, tgt_ref, mask_ref, out_ref, skey_ref, mk_ref, acc_ref, thr_ref):
    s = pl.program_id(0)
    c = pl.program_id(1)

    @pl.when((s == 0) & (c == 0))
    def _init():
        acc_ref[0] = 0.0

    @pl.when(c == 0)
    def _stage_threshold():
        def build_chunk(i, carry):
            pos_c, neg_c = carry
            rows = pl.ds(i * _ROWS, _ROWS)
            t = pred_ref[0, 0, rows, :]
            g = tgt_ref[0, 0, rows, :]
            m = mask_ref[0, rows, :]
            neg = g <= 0.5
            skey_ref[rows, :] = jnp.where(neg, _float_key(t), _INT_MIN)
            mk_ref[rows, :] = ((t > 0.0) & (m > 0.5)).astype(jnp.float32)
            pos_c += jnp.sum(((g > 0.5) & (m > 0.5)).astype(jnp.int32))
            neg_c += jnp.sum(neg.astype(jnp.int32))
            return pos_c, neg_c

        pos_num, neg_total = jax.lax.fori_loop(
            0, 512 // _ROWS, build_chunk, (jnp.int32(0), jnp.int32(0))
        )
        neg_num = jnp.minimum(pos_num * 3, neg_total)

        def bit_step(_, carry):
            result, bit = carry
            cand = result + bit

            def count_chunk(i, acc):
                sk = skey_ref[pl.ds(i * _ROWS, _ROWS), :]
                return acc + jnp.sum((sk >= cand).astype(jnp.int32))

            cnt = jax.lax.fori_loop(0, 512 // _ROWS, count_chunk, jnp.int32(0))
            return jnp.where(cnt >= neg_num, cand, result), bit >> 1

        sstar, _ = jax.lax.fori_loop(
            0, 31, bit_step, (jnp.int32(_INT_MIN), jnp.int32(1 << 30))
        )
        thr_ref[0] = sstar
        thr_ref[1] = neg_num

    @pl.when(c == 1)
    def _stage_text_dice():
        sstar = thr_ref[0]
        neg_num = thr_ref[1]

        def dice_chunk(i, carry):
            a, b, cc = carry
            rows = pl.ds(i * _ROWS, _ROWS)
            t = pred_ref[0, 0, rows, :]
            g = tgt_ref[0, 0, rows, :]
            m = mask_ref[0, rows, :]
            sel = ((_float_key(t) >= sstar) | (g > 0.5)) & (m > 0.5)
            meff = jnp.where(neg_num == 0, m, sel.astype(jnp.float32))
            m2 = meff * meff
            sig = 1.0 / (1.0 + jnp.exp(-t))
            a += jnp.sum(sig * g * m2)
            b += jnp.sum(sig * sig * m2)
            cc += jnp.sum(g * g * m2)
            return a, b, cc

        a, b, cc = jax.lax.fori_loop(
            0, 512 // _ROWS, dice_chunk,
            (jnp.float32(0), jnp.float32(0), jnp.float32(0)),
        )
        d = 2.0 * a / (b + cc + _EPS)
        acc_ref[0] += (0.7 / 8.0) * d

    @pl.when(c >= 2)
    def _stage_kernel_dice():
        def dice_chunk(i, carry):
            a, b, cc = carry
            rows = pl.ds(i * _ROWS, _ROWS)
            t = pred_ref[0, 0, rows, :]
            g = tgt_ref[0, 0, rows, :]
            mk = mk_ref[rows, :]
            sig = 1.0 / (1.0 + jnp.exp(-t))
            a += jnp.sum(sig * g * mk)
            b += jnp.sum(sig * sig * mk)
            cc += jnp.sum(g * g * mk)
            return a, b, cc

        a, b, cc = jax.lax.fori_loop(
            0, 512 // _ROWS, dice_chunk,
            (jnp.float32(0), jnp.float32(0), jnp.float32(0)),
        )
        d = 2.0 * a / (b + cc + _EPS)
        acc_ref[0] += (0.3 / 48.0) * d

    out_ref[0, 0] = 1.0 - acc_ref[0]


def kernel(model_predict, target, training_masks):
    out = pl.pallas_call(
        _body,
        grid=(8, 8),
        in_specs=[
            pl.BlockSpec(
                (1, 1, 512, 512), lambda s, c: (s, jnp.maximum(c - 1, 0), 0, 0)
            ),
            pl.BlockSpec(
                (1, 1, 512, 512), lambda s, c: (s, jnp.maximum(c - 1, 0), 0, 0)
            ),
            pl.BlockSpec((1, 512, 512), lambda s, c: (s, 0, 0)),
        ],
        out_specs=pl.BlockSpec((1, 1), lambda s, c: (0, 0), memory_space=pltpu.SMEM),
        out_shape=jax.ShapeDtypeStruct((1, 1), jnp.float32),
        scratch_shapes=[
            pltpu.VMEM((512, 512), jnp.int32),
            pltpu.VMEM((512, 512), jnp.float32),
            pltpu.SMEM((1,), jnp.float32),
            pltpu.SMEM((2,), jnp.int32),
        ],
    )(model_predict, target, training_masks)
    return out[0, 0]


# speedup vs baseline: 33.9110x; 4.9895x over previous
import jax
import jax.numpy as jnp
import numpy as np
from jax.experimental import pallas as pl
from jax.experimental.pallas import tpu as pltpu

_EPS = float(np.spacing(1.0))
_INT_MIN = np.int32(-2147483648)
_INT_MAX = np.int32(2147483647)
_ROWS = 64
_NCHUNK = 512 // _ROWS


def _float_key(t):
    ti = jax.lax.bitcast_convert_type(t, jnp.int32)
    return jnp.where(ti < 0, _INT_MIN - ti, ti)


def _rowsum8(x):
    r = x[0:8]
    for j in range(1, 8):
        r = r + x[8 * j : 8 * j + 8]
    return r


def _rowmin8(x):
    r = x[0:8]
    for j in range(1, 8):
        r = jnp.minimum(r, x[8 * j : 8 * j + 8])
    return r


def _body(pred_ref, tgt_ref, mask_ref, out_ref, skey_ref, mk_ref, acc_ref, thr_ref):
    s = pl.program_id(0)
    c = pl.program_id(1)

    @pl.when((s == 0) & (c == 0))
    def _init():
        acc_ref[0] = 0.0

    @pl.when(c == 0)
    def _stage_threshold():
        zero8 = jnp.zeros((8, 512), jnp.int32)
        pos_v = zero8
        neg_v = zero8
        for i in range(_NCHUNK):
            rows = pl.ds(i * _ROWS, _ROWS)
            t = pred_ref[0, 0, rows, :]
            g = tgt_ref[0, 0, rows, :]
            m = mask_ref[0, rows, :]
            neg = g <= 0.5
            skey_ref[rows, :] = jnp.where(neg, _float_key(t), _INT_MIN)
            mk_ref[rows, :] = ((t > 0.0) & (m > 0.5)).astype(jnp.float32)
            pos_v += _rowsum8(((g > 0.5) & (m > 0.5)).astype(jnp.int32))
            neg_v += _rowsum8(neg.astype(jnp.int32))
        pos_num = jnp.sum(pos_v)
        neg_total = jnp.sum(neg_v)
        neg_num = jnp.minimum(pos_num * 3, neg_total)

        def _min_path(_):
            mn = jnp.full((8, 512), _INT_MAX, jnp.int32)
            for i in range(_NCHUNK):
                sk = skey_ref[pl.ds(i * _ROWS, _ROWS), :]
                mn = jnp.minimum(mn, _rowmin8(jnp.where(sk == _INT_MIN, _INT_MAX, sk)))
            return jnp.min(mn)

        def _search_path(_):
            def count_ge(cand):
                cnt_v = jnp.zeros((8, 512), jnp.int32)
                for i in range(_NCHUNK):
                    sk = skey_ref[pl.ds(i * _ROWS, _ROWS), :]
                    cnt_v += _rowsum8((sk >= cand).astype(jnp.int32))
                return jnp.sum(cnt_v)

            start = jnp.where(
                count_ge(jnp.int32(0)) >= neg_num, jnp.int32(0), jnp.int32(_INT_MIN)
            )

            def bit_step(_, carry):
                result, bit = carry
                cand = result + bit
                cnt = count_ge(cand)
                return jnp.where(cnt >= neg_num, cand, result), bit >> 1

            res, _ = jax.lax.fori_loop(0, 31, bit_step, (start, jnp.int32(1 << 30)))
            return res

        sstar = jax.lax.cond(
            neg_num == neg_total, _min_path, _search_path, jnp.int32(0)
        )
        thr_ref[0] = sstar
        thr_ref[1] = neg_num

    @pl.when(c == 1)
    def _stage_text_dice():
        sstar = thr_ref[0]
        neg_num = thr_ref[1]
        zero8 = jnp.zeros((8, 512), jnp.float32)
        a_v, b_v, c_v = zero8, zero8, zero8
        for i in range(_NCHUNK):
            rows = pl.ds(i * _ROWS, _ROWS)
            t = pred_ref[0, 0, rows, :]
            g = tgt_ref[0, 0, rows, :]
            m = mask_ref[0, rows, :]
            sel = ((_float_key(t) >= sstar) | (g > 0.5)) & (m > 0.5)
            meff = jnp.where(neg_num == 0, m, sel.astype(jnp.float32))
            m2 = meff * meff
            sig = 1.0 / (1.0 + jnp.exp(-t))
            a_v += _rowsum8(sig * g * m2)
            b_v += _rowsum8(sig * sig * m2)
            c_v += _rowsum8(g * g * m2)
        a, b, cc = jnp.sum(a_v), jnp.sum(b_v), jnp.sum(c_v)
        d = 2.0 * a / (b + cc + _EPS)
        acc_ref[0] += (0.7 / 8.0) * d

    @pl.when(c >= 2)
    def _stage_kernel_dice():
        zero8 = jnp.zeros((8, 512), jnp.float32)
        a_v, b_v, c_v = zero8, zero8, zero8
        for i in range(_NCHUNK):
            rows = pl.ds(i * _ROWS, _ROWS)
            t = pred_ref[0, 0, rows, :]
            g = tgt_ref[0, 0, rows, :]
            mk = mk_ref[rows, :]
            sig = 1.0 / (1.0 + jnp.exp(-t))
            a_v += _rowsum8(sig * g * mk)
            b_v += _rowsum8(sig * sig * mk)
            c_v += _rowsum8(g * g * mk)
        a, b, cc = jnp.sum(a_v), jnp.sum(b_v), jnp.sum(c_v)
        d = 2.0 * a / (b + cc + _EPS)
        acc_ref[0] += (0.3 / 48.0) * d

    out_ref[0, 0] = 1.0 - acc_ref[0]


def kernel(model_predict, target, training_masks):
    out = pl.pallas_call(
        _body,
        grid=(8, 8),
        in_specs=[
            pl.BlockSpec(
                (1, 1, 512, 512), lambda s, c: (s, jnp.maximum(c - 1, 0), 0, 0)
            ),
            pl.BlockSpec(
                (1, 1, 512, 512), lambda s, c: (s, jnp.maximum(c - 1, 0), 0, 0)
            ),
            pl.BlockSpec((1, 512, 512), lambda s, c: (s, 0, 0)),
        ],
        out_specs=pl.BlockSpec((1, 1), lambda s, c: (0, 0), memory_space=pltpu.SMEM),
        out_shape=jax.ShapeDtypeStruct((1, 1), jnp.float32),
        scratch_shapes=[
            pltpu.VMEM((512, 512), jnp.int32),
            pltpu.VMEM((512, 512), jnp.float32),
            pltpu.SMEM((1,), jnp.float32),
            pltpu.SMEM((2,), jnp.int32),
        ],
    )(model_predict, target, training_masks)
    return out[0, 0]
